# baseline (speedup 1.0000x reference)
import functools

import jax
import jax.numpy as jnp
from jax import lax
from jax.experimental import pallas as pl
from jax.experimental.pallas import tpu as pltpu
from jax.experimental.pallas import tpu_sc as plsc

_B = 16384
_D = 64
_L = 16


@functools.cache
def _build():
    info = plsc.get_sparse_core_info()
    nc, ns = info.num_cores, info.num_subcores
    nw = nc * ns
    bpw = _B // nw

    mesh = plsc.VectorSubcoreMesh(core_axis_name="c", subcore_axis_name="s")

    @functools.partial(
        pl.kernel,
        mesh=mesh,
        compiler_params=pltpu.CompilerParams(needs_layout_passes=False),
        out_type=jax.ShapeDtypeStruct((_B,), jnp.float32),
        scratch_types=[
            pltpu.VMEM((bpw,), jnp.int32),
            pltpu.VMEM((bpw,), jnp.int32),
            pltpu.VMEM((bpw // 2, _D), jnp.float32),
            pltpu.VMEM((bpw // 2, _D), jnp.float32),
            pltpu.VMEM((bpw,), jnp.float32),
            pltpu.VMEM((bpw,), jnp.float32),
            pltpu.VMEM((bpw,), jnp.float32),
            pltpu.SemaphoreType.DMA,
            pltpu.SemaphoreType.DMA,
            pltpu.SemaphoreType.DMA,
        ],
    )
    def sc_kernel(uids_hbm, aids_hbm, uemb_hbm, aemb_hbm, ub_hbm, ab_hbm,
                  out_hbm, uidx, aidx, urows, arows, ubv, abv, outv,
                  sem_rows, sem_b0, sem_b1):
        wid = lax.axis_index("s") * nc + lax.axis_index("c")
        base = wid * bpw
        pltpu.sync_copy(uids_hbm.at[pl.ds(base, bpw)], uidx)
        pltpu.sync_copy(aids_hbm.at[pl.ds(base, bpw)], aidx)
        cb0 = pltpu.async_copy(ub_hbm.at[uidx], ubv, sem_b0)
        cb1 = pltpu.async_copy(ab_hbm.at[aidx], abv, sem_b1)

        cb0.wait()
        cb1.wait()

        lane = lax.iota(jnp.int32, _L)
        chunk = bpw // 2

        for half in range(2):
            off = half * chunk

            def issue_body(g, carry, off=off):
                uvec = uidx[pl.ds(off + g * _L, _L)]
                avec = aidx[pl.ds(off + g * _L, _L)]
                for r in range(_L):
                    i = g * _L + r
                    pltpu.async_copy(uemb_hbm.at[uvec[r]], urows.at[i],
                                     sem_rows)
                    pltpu.async_copy(aemb_hbm.at[avec[r]], arows.at[i],
                                     sem_rows)
                return carry

            lax.fori_loop(0, chunk // _L, issue_body, 0)

            def drain_body(i, carry):
                pltpu.make_async_copy(uemb_hbm.at[0], urows.at[i],
                                     sem_rows).wait()
                pltpu.make_async_copy(aemb_hbm.at[0], arows.at[i],
                                     sem_rows).wait()
                return carry

            lax.fori_loop(0, chunk, drain_body, 0)

            def dot_body(g, carry, off=off):
                sl = pl.ds(off + g * _L, _L)
                acc = ubv[sl] + abv[sl]
                for r in range(_L):
                    i = g * _L + r
                    p = urows[i, pl.ds(0, _L)] * arows[i, pl.ds(0, _L)]
                    for j in range(1, _D // _L):
                        p = p + (urows[i, pl.ds(j * _L, _L)] *
                                 arows[i, pl.ds(j * _L, _L)])
                    acc = jnp.where(lane == r, jnp.sum(p) + acc, acc)
                outv[sl] = acc
                return carry

            lax.fori_loop(0, chunk // _L, dot_body, 0)

        pltpu.sync_copy(outv, out_hbm.at[pl.ds(base, bpw)])

    return sc_kernel


def kernel(userIds, animeIds, user_embeddings, anime_embeddings,
           user_biases, anime_biases):
    uids = userIds.astype(jnp.int32)
    aids = animeIds.astype(jnp.int32)
    ub = user_biases.reshape((-1,))
    ab = anime_biases.reshape((-1,))
    return _build()(uids, aids, user_embeddings, anime_embeddings, ub, ab)
